# Initial kernel scaffold; baseline (speedup 1.0000x reference)
import jax
import jax.numpy as jnp
from jax.experimental import pallas as pl


def kernel(IX, weight):
    raise NotImplementedError("write your pallas kernel here")


# speedup vs baseline: 1.4419x; 1.4419x over previous
import functools

import jax
import jax.numpy as jnp
from jax import lax
from jax.experimental import pallas as pl
from jax.experimental.pallas import tpu as pltpu
from jax.experimental.pallas import tpu_sc as plsc

B = 16384 * 26
D = 32
NC = 2
NS = 16
NW = NC * NS
BPW = B // NW
C = 128
NCHUNK = BPW // C

_mesh = plsc.VectorSubcoreMesh(core_axis_name="c", subcore_axis_name="s")


@functools.partial(
    pl.kernel,
    mesh=_mesh,
    out_type=jax.ShapeDtypeStruct((B, D), jnp.float32),
    scratch_types=[
        pltpu.VMEM((BPW,), jnp.int32),
        pltpu.VMEM((C, D), jnp.float32),
        pltpu.SemaphoreType.DMA,
    ],
    compiler_params=pltpu.CompilerParams(use_tc_tiling_on_sc=False),
)
def _gather_kernel(ix_hbm, w_hbm, out_hbm, idx_v, rows_v, sem):
    wid = lax.axis_index("s") * NC + lax.axis_index("c")
    base = wid * BPW
    pltpu.sync_copy(ix_hbm.at[pl.ds(base, BPW)], idx_v)

    def body(c, carry):
        off = pl.multiple_of(c * C, 8)
        pltpu.async_copy(w_hbm.at[idx_v.at[pl.ds(off, C)]], rows_v, sem).wait()
        pltpu.sync_copy(rows_v, out_hbm.at[pl.ds(base + off, C)])
        return carry

    lax.fori_loop(0, NCHUNK, body, 0)


def kernel(IX, weight):
    flat = IX.reshape(-1).astype(jnp.int32)
    out = _gather_kernel(flat, weight)
    return out.reshape(IX.shape + (weight.shape[1],))


# speedup vs baseline: 1.5400x; 1.0680x over previous
import functools

import jax
import jax.numpy as jnp
from jax import lax
from jax.experimental import pallas as pl
from jax.experimental.pallas import tpu as pltpu
from jax.experimental.pallas import tpu_sc as plsc

B = 16384 * 26
D = 32
NC = 2
NS = 16
NW = NC * NS
BPW = B // NW
C = 512
NCHUNK = BPW // C

_mesh = plsc.VectorSubcoreMesh(core_axis_name="c", subcore_axis_name="s")


@functools.partial(
    pl.kernel,
    mesh=_mesh,
    out_type=jax.ShapeDtypeStruct((B, D), jnp.float32),
    scratch_types=[
        pltpu.VMEM((BPW,), jnp.int32),
        pltpu.VMEM((C, D), jnp.float32),
        pltpu.SemaphoreType.DMA,
    ],
    compiler_params=pltpu.CompilerParams(use_tc_tiling_on_sc=False),
)
def _gather_kernel(ix_hbm, w_hbm, out_hbm, idx_v, rows_v, sem):
    wid = lax.axis_index("s") * NC + lax.axis_index("c")
    base = wid * BPW
    pltpu.sync_copy(ix_hbm.at[pl.ds(base, BPW)], idx_v)

    def body(c, carry):
        off = pl.multiple_of(c * C, 8)
        pltpu.async_copy(w_hbm.at[idx_v.at[pl.ds(off, C)]], rows_v, sem).wait()
        pltpu.sync_copy(rows_v, out_hbm.at[pl.ds(base + off, C)])
        return carry

    lax.fori_loop(0, NCHUNK, body, 0)


def kernel(IX, weight):
    flat = IX.reshape(-1).astype(jnp.int32)
    out = _gather_kernel(flat, weight)
    return out.reshape(IX.shape + (weight.shape[1],))


# speedup vs baseline: 1.5815x; 1.0270x over previous
import functools

import jax
import jax.numpy as jnp
from jax import lax
from jax.experimental import pallas as pl
from jax.experimental.pallas import tpu as pltpu
from jax.experimental.pallas import tpu_sc as plsc

B = 16384 * 26
D = 32
NC = 2
NS = 16
NW = NC * NS
BPW = B // NW
C = 416
NCHUNK = BPW // C
NBUF = 8
K = 4
NROUNDS = NCHUNK // NBUF

_mesh = plsc.VectorSubcoreMesh(core_axis_name="c", subcore_axis_name="s")


@functools.partial(
    pl.kernel,
    mesh=_mesh,
    out_type=jax.ShapeDtypeStruct((B, D), jnp.float32),
    scratch_types=[
        pltpu.VMEM((BPW,), jnp.int32),
        [pltpu.VMEM((C, D), jnp.float32) for _ in range(NBUF)],
        [pltpu.SemaphoreType.DMA for _ in range(NBUF)],
        [pltpu.SemaphoreType.DMA for _ in range(NBUF)],
    ],
    compiler_params=pltpu.CompilerParams(use_tc_tiling_on_sc=False),
)
def _gather_kernel(ix_hbm, w_hbm, out_hbm, idx_v, bufs, gsems, wsems):
    wid = lax.axis_index("s") * NC + lax.axis_index("c")
    base = wid * BPW
    pltpu.sync_copy(ix_hbm.at[pl.ds(base, BPW)], idx_v)

    def fire_gather(c, b):
        off = pl.multiple_of(c * C, 8)
        pltpu.async_copy(w_hbm.at[idx_v.at[pl.ds(off, C)]], bufs[b], gsems[b])

    def wait_gather_fire_write(c, b):
        off = pl.multiple_of(c * C, 8)
        pltpu.make_async_copy(w_hbm.at[idx_v.at[pl.ds(off, C)]], bufs[b],
                              gsems[b]).wait()
        pltpu.async_copy(bufs[b], out_hbm.at[pl.ds(base + off, C)], wsems[b])

    def wait_write(c, b):
        off = pl.multiple_of(c * C, 8)
        pltpu.make_async_copy(bufs[b], out_hbm.at[pl.ds(base + off, C)],
                              wsems[b]).wait()

    for c in range(K):
        fire_gather(c, c % NBUF)

    for b in range(NBUF):
        wait_gather_fire_write(b, b)
        if b + K >= NBUF:
            wait_write(b + K - NBUF, (b + K) % NBUF)
        fire_gather(b + K, (b + K) % NBUF)

    def round_body(r, carry):
        g = pl.multiple_of(r * NBUF, NBUF)
        for b in range(NBUF):
            c = g + b
            wait_gather_fire_write(c, b)
            wait_write(c + K - NBUF, (b + K) % NBUF)
            fire_gather(c + K, (b + K) % NBUF)
        return carry

    lax.fori_loop(1, NROUNDS - 1, round_body, 0)

    g = (NROUNDS - 1) * NBUF
    for b in range(NBUF):
        c = g + b
        wait_gather_fire_write(c, b)
        if b < NBUF - K:
            wait_write(c + K - NBUF, (b + K) % NBUF)
            fire_gather(c + K, (b + K) % NBUF)

    for c in range(NCHUNK - NBUF, NCHUNK):
        wait_write(c, c % NBUF)


def kernel(IX, weight):
    flat = IX.reshape(-1).astype(jnp.int32)
    out = _gather_kernel(flat, weight)
    return out.reshape(IX.shape + (weight.shape[1],))


# speedup vs baseline: 1.6720x; 1.0572x over previous
import functools

import jax
import jax.numpy as jnp
from jax import lax
from jax.experimental import pallas as pl
from jax.experimental.pallas import tpu as pltpu
from jax.experimental.pallas import tpu_sc as plsc

B = 16384 * 26
D = 32
NC = 2
NS = 16
NW = NC * NS
BPW = B // NW
C = 416
NCHUNK = BPW // C
NBUF = 8
K = 4
NROUNDS = NCHUNK // NBUF

_mesh = plsc.VectorSubcoreMesh(core_axis_name="c", subcore_axis_name="s")


@functools.partial(
    pl.kernel,
    mesh=_mesh,
    out_type=jax.ShapeDtypeStruct((B, D), jnp.float32),
    scratch_types=[
        pltpu.VMEM((BPW,), jnp.int32),
        [pltpu.VMEM((C, D), jnp.float32) for _ in range(NBUF)],
        [pltpu.SemaphoreType.DMA for _ in range(NBUF)],
        [pltpu.SemaphoreType.DMA for _ in range(NBUF)],
    ],
    compiler_params=pltpu.CompilerParams(use_tc_tiling_on_sc=False),
)
def _gather_kernel(ix_hbm, w_hbm, out_hbm, idx_v, bufs, gsems, wsems):
    wid = lax.axis_index("s") * NC + lax.axis_index("c")
    base = wid * BPW
    pltpu.sync_copy(ix_hbm.at[pl.ds(base, BPW)], idx_v)

    def fire_gather(c, b):
        off = pl.multiple_of(c * C, 8)
        pltpu.async_copy(w_hbm.at[idx_v.at[pl.ds(off, C)]], bufs[b], gsems[b])

    def wait_gather_fire_write(c, b):
        off = pl.multiple_of(c * C, 8)
        pltpu.make_async_copy(w_hbm.at[idx_v.at[pl.ds(off, C)]], bufs[b],
                              gsems[b]).wait()
        pltpu.async_copy(bufs[b], out_hbm.at[pl.ds(base + off, C)], wsems[b])

    def wait_write(c, b):
        off = pl.multiple_of(c * C, 8)
        pltpu.make_async_copy(bufs[b], out_hbm.at[pl.ds(base + off, C)],
                              wsems[b]).wait()

    for c in range(K):
        fire_gather(c, c % NBUF)

    for b in range(NBUF):
        wait_gather_fire_write(b, b)
        if b + K >= NBUF:
            wait_write(b + K - NBUF, (b + K) % NBUF)
        fire_gather(b + K, (b + K) % NBUF)

    def round_body(r, carry):
        g = pl.multiple_of(r * NBUF, NBUF)
        for b in range(NBUF):
            c = g + b
            wait_gather_fire_write(c, b)
            wait_write(c + K - NBUF, (b + K) % NBUF)
            fire_gather(c + K, (b + K) % NBUF)
        return carry

    lax.fori_loop(1, NROUNDS - 1, round_body, 0)

    g = (NROUNDS - 1) * NBUF
    for b in range(NBUF):
        c = g + b
        wait_gather_fire_write(c, b)
        if b < NBUF - K:
            wait_write(c + K - NBUF, (b + K) % NBUF)
            fire_gather(c + K, (b + K) % NBUF)

    for c in range(NCHUNK - NBUF, NCHUNK):
        wait_write(c, c % NBUF)


def kernel(IX, weight):
    nb, nf = IX.shape
    flat = IX.T.reshape(-1).astype(jnp.int32)
    out = _gather_kernel(flat, weight)
    return out.reshape(nf, nb, weight.shape[1]).transpose(1, 0, 2)
